# Initial kernel scaffold; baseline (speedup 1.0000x reference)
import jax
import jax.numpy as jnp
from jax.experimental import pallas as pl


def kernel(hiddens, keys, encoded_sents, indices, U, V, W):
    raise NotImplementedError("write your pallas kernel here")


# speedup vs baseline: 1.1496x; 1.1496x over previous
import functools

import jax
import jax.numpy as jnp
from jax.experimental import pallas as pl
from jax.experimental.pallas import tpu as pltpu

B, E, D, P = 16384, 64, 64, 4096
R = 256
NB = B // R


def _body(sidx_ref, order_ref, starts_ref,
          h_ref, k_ref, enc_ref, u_ref, v_ref, w_ref,
          out_ref,
          encw_ref):
    i = pl.program_id(0)

    @pl.when(i == 0)
    def _():
        encw_ref[...] = jnp.dot(enc_ref[...], w_ref[...],
                                preferred_element_type=jnp.float32)

    out_ref[...] = h_ref[...]

    u = u_ref[...]
    v = v_ref[...]
    s = starts_ref[i]
    e = starts_ref[i + 1]

    def step(j, _):
        r = sidx_ref[j] - i * R
        o = order_ref[j]
        h = h_ref[r]
        k = k_ref[r]
        ec = enc_ref[o]
        ecw = encw_ref[o]
        gate = jax.nn.sigmoid(jnp.sum((h + k) * ec[None, :], axis=1))
        ht = jax.nn.relu(
            jnp.dot(h, u, preferred_element_type=jnp.float32)
            + jnp.dot(k, v, preferred_element_type=jnp.float32)
            + ecw[None, :])
        out_ref[r] = out_ref[r] + gate[:, None] * ht
        return _

    jax.lax.fori_loop(s, e, step, None)

    x = out_ref[...]
    sq = jnp.sum(x * x, axis=2, keepdims=True)
    out_ref[...] = x * jax.lax.rsqrt(jnp.maximum(sq, 1e-12))


@functools.partial(jax.jit, static_argnames=("interpret",))
def kernel(hiddens, keys, encoded_sents, indices, U, V, W, interpret=False):
    order = jnp.argsort(indices).astype(jnp.int32)
    sidx = indices[order].astype(jnp.int32)
    starts = jnp.searchsorted(
        sidx, (jnp.arange(NB + 1) * R).astype(jnp.int32), side="left"
    ).astype(jnp.int32)

    grid_spec = pltpu.PrefetchScalarGridSpec(
        num_scalar_prefetch=3,
        grid=(NB,),
        in_specs=[
            pl.BlockSpec((R, E, D), lambda i, *_: (i, 0, 0)),
            pl.BlockSpec((R, E, D), lambda i, *_: (i, 0, 0)),
            pl.BlockSpec((P, D), lambda i, *_: (0, 0)),
            pl.BlockSpec((D, D), lambda i, *_: (0, 0)),
            pl.BlockSpec((D, D), lambda i, *_: (0, 0)),
            pl.BlockSpec((D, D), lambda i, *_: (0, 0)),
        ],
        out_specs=pl.BlockSpec((R, E, D), lambda i, *_: (i, 0, 0)),
        scratch_shapes=[pltpu.VMEM((P, D), jnp.float32)],
    )
    return pl.pallas_call(
        _body,
        grid_spec=grid_spec,
        out_shape=jax.ShapeDtypeStruct((B, E, D), jnp.float32),
        interpret=interpret,
    )(sidx, order, starts, hiddens, keys, encoded_sents, U, V, W)
